# Initial kernel scaffold; baseline (speedup 1.0000x reference)
import jax
import jax.numpy as jnp
from jax.experimental import pallas as pl


def kernel(x, params):
    raise NotImplementedError("write your pallas kernel here")


# speedup vs baseline: 45.2235x; 45.2235x over previous
import functools

import jax
import jax.numpy as jnp
import numpy as np
from jax.experimental import pallas as pl
from jax.experimental.pallas import tpu as pltpu

_EPS = 1e-5
_N = 1024
_B = 16
_INV2SIG2 = 12.5


def _mm(a, b):
    return jax.lax.dot_general(a, b, (((1,), (0,)), ((), ())),
                               preferred_element_type=jnp.float32)


def _relu(v):
    return jnp.maximum(v, 0.0)


def _gather(P, M):
    hi = M.astype(jnp.bfloat16)
    lo = (M - hi.astype(jnp.float32)).astype(jnp.bfloat16)
    return _mm(P, hi) + _mm(P, lo)


def _mesh_body(face_ref, nbr_ref,
               sp1W, sp1b, sp2W, sp2b,
               r1W0, r1W1, r1W2, r1b, r2W, r2b,
               f1W, f1b, f2W, f2b,
               Wm, wsq, fks, fkb,
               s1Wa, s1Wb, s1Wc, s1b, s2W, s2b,
               c1Wa, c1Wb, c1b, a1W, a1b,
               c2Wa, c2Wb, c2b, a2W, a2b,
               fuWa, fuWb, fub,
               caWa, caWb, caWc, cab,
               out_ref):
    f = face_ref[0]
    nbr = nbr_ref[0]

    lane = jax.lax.broadcasted_iota(jnp.int32, (_N, _N), 1)
    P0 = (lane == nbr[:, 0:1]).astype(jnp.bfloat16)
    P1 = (lane == nbr[:, 1:2]).astype(jnp.bfloat16)
    P2 = (lane == nbr[:, 2:3]).astype(jnp.bfloat16)

    centers = f[:, 0:3]
    corners = f[:, 3:12] - jnp.concatenate([centers, centers, centers], axis=1)
    nrm = f[:, 12:15]

    sf0 = _relu(_mm(_relu(_mm(centers, sp1W[...]) + sp1b[...]), sp2W[...]) + sp2b[...])

    def rmlp(W1):
        z = _relu(_mm(corners, W1[...]) + r1b[...])
        return _relu(_mm(z, r2W[...]) + r2b[...])
    fea = (rmlp(r1W0) + rmlp(r1W1) + rmlp(r1W2)) * (1.0 / 3.0)
    frc = _relu(_mm(_relu(_mm(fea, f1W[...]) + f1b[...]), f2W[...]) + f2b[...])

    acc = jnp.zeros((_N, 256), jnp.float32)
    for F in (nrm, _gather(P0, nrm), _gather(P1, nrm), _gather(P2, nrm)):
        fsq = jnp.sum(F * F, axis=1, keepdims=True)
        d = fsq + wsq[...] - 2.0 * _mm(F, Wm[...])
        acc = acc + jnp.exp(-_INV2SIG2 * d)
    fkc = (acc[:, 0:64] + acc[:, 64:128] + acc[:, 128:192] + acc[:, 192:256]) * (1.0 / 16.0)
    fkc = _relu(fkc * fks[...] + fkb[...])

    st = _relu(_mm(frc, s1Wa[...]) + _mm(fkc, s1Wb[...]) + _mm(nrm, s1Wc[...]) + s1b[...])
    st0 = _relu(_mm(st, s2W[...]) + s2b[...])

    sp1 = _relu(_mm(sf0, c1Wa[...]) + _mm(st0, c1Wb[...]) + c1b[...])
    g = jnp.maximum(jnp.maximum(st0, _gather(P0, st0)),
                    jnp.maximum(_gather(P1, st0), _gather(P2, st0)))
    st1 = _relu(_mm(g, a1W[...]) + a1b[...])

    sp2 = _relu(_mm(sp1, c2Wa[...]) + _mm(st1, c2Wb[...]) + c2b[...])
    g = jnp.maximum(jnp.maximum(st1, _gather(P0, st1)),
                    jnp.maximum(_gather(P1, st1), _gather(P2, st1)))
    st2 = _relu(_mm(g, a2W[...]) + a2b[...])

    sp3 = _relu(_mm(sp2, fuWa[...]) + _mm(st2, fuWb[...]) + fub[...])
    fea2 = _relu(_mm(sp1, caWa[...]) + _mm(sp2, caWb[...]) + _mm(sp3, caWc[...]) + cab[...])
    out_ref[0] = jnp.max(fea2, axis=0, keepdims=True)


def _head_body(fm_ref, W1, b1, W2, b2, W3, b3, y_ref, hn_ref):
    h = _relu(_mm(fm_ref[...], W1[...]) + b1[...])
    h = _relu(_mm(h, W2[...]) + b2[...])
    y_ref[...] = _mm(h, W3[...]) + b3[...]
    hn_ref[...] = h * jax.lax.rsqrt(jnp.sum(h * h))


def _fold(p, name):
    s = p[name + '_g'] * (1.0 / np.sqrt(1.0 + _EPS))
    W = (p[name + '_W'] * s[:, None]).T
    b = (p[name + '_b'] * s + p[name + '_be'])[None, :]
    return W, b


@jax.jit
def kernel(x, params):
    p = params
    face = x[..., :15]
    nbr = x[..., 15:18].astype(jnp.int32)

    sp1W, sp1b = _fold(p, 'sp1')
    sp2W, sp2b = _fold(p, 'sp2')
    r1W, r1b = _fold(p, 'rot1')
    z3 = jnp.zeros((3, 32), jnp.float32)
    r1W0 = jnp.concatenate([r1W, z3], axis=0)
    r1W1 = jnp.concatenate([z3, r1W], axis=0)
    r1W2 = jnp.concatenate([r1W[3:6], z3, r1W[0:3]], axis=0)
    r2W, r2b = _fold(p, 'rot2')
    f1W, f1b = _fold(p, 'fus1')
    f2W, f2b = _fold(p, 'fus2')

    alpha, beta = p['fkc_alpha'], p['fkc_beta']
    w3 = jnp.concatenate([jnp.sin(alpha) * jnp.cos(beta),
                          jnp.sin(alpha) * jnp.sin(beta),
                          jnp.cos(alpha)], axis=0)
    Wm = jnp.transpose(w3, (0, 2, 1)).reshape(3, 256)
    wsq = jnp.sum(Wm * Wm, axis=0)[None, :]
    fks = (p['fkc_g'] * (1.0 / np.sqrt(1.0 + _EPS)))[None, :]
    fkb = p['fkc_be'][None, :]

    s1W, s1b = _fold(p, 'str1')
    s1Wa, s1Wb, s1Wc = s1W[0:64], s1W[64:128], s1W[128:131]
    s2W, s2b = _fold(p, 'str2')
    c1W, c1b = _fold(p, 'mc1_comb')
    c1Wa, c1Wb = c1W[0:64], c1W[64:195]
    a1W, a1b = _fold(p, 'mc1_agg')
    c2W, c2b = _fold(p, 'mc2_comb')
    c2Wa, c2Wb = c2W[0:256], c2W[256:512]
    a2W, a2b = _fold(p, 'mc2_agg')
    fuW, fub = _fold(p, 'fusion')
    fuWa, fuWb = fuW[0:512], fuW[512:1024]
    caW, cab = _fold(p, 'cat')
    caWa, caWb, caWc = caW[0:256], caW[256:768], caW[768:1792]

    weights = [sp1W, sp1b, sp2W, sp2b,
               r1W0, r1W1, r1W2, r1b, r2W, r2b,
               f1W, f1b, f2W, f2b,
               Wm, wsq, fks, fkb,
               s1Wa, s1Wb, s1Wc, s1b, s2W, s2b,
               c1Wa, c1Wb, c1b, a1W, a1b,
               c2Wa, c2Wb, c2b, a2W, a2b,
               fuWa, fuWb, fub,
               caWa, caWb, caWc, cab]

    w_specs = [pl.BlockSpec(w.shape, functools.partial(lambda b, _s: (0,) * len(_s), _s=w.shape))
               for w in weights]

    fm = pl.pallas_call(
        _mesh_body,
        grid=(_B,),
        in_specs=[pl.BlockSpec((1, _N, 15), lambda b: (b, 0, 0)),
                  pl.BlockSpec((1, _N, 3), lambda b: (b, 0, 0))] + w_specs,
        out_specs=pl.BlockSpec((1, 1, 1024), lambda b: (b, 0, 0)),
        out_shape=jax.ShapeDtypeStruct((_B, 1, 1024), jnp.float32),
        compiler_params=pltpu.CompilerParams(dimension_semantics=("arbitrary",)),
    )(face, nbr, *weights)
    fm = fm.reshape(_B, 1024)

    hw = [p['l1_W'].T, p['l1_b'][None, :],
          p['l2_W'].T, p['l2_b'][None, :],
          p['l3_W'].T, p['l3_b'][None, :]]
    y, hn = pl.pallas_call(
        _head_body,
        in_specs=[pl.BlockSpec(fm.shape, lambda: (0, 0))] +
                 [pl.BlockSpec(w.shape, lambda: (0, 0)) for w in hw],
        out_specs=[pl.BlockSpec((_B, 40), lambda: (0, 0)),
                   pl.BlockSpec((_B, 256), lambda: (0, 0))],
        out_shape=[jax.ShapeDtypeStruct((_B, 40), jnp.float32),
                   jax.ShapeDtypeStruct((_B, 256), jnp.float32)],
    )(fm, *hw)
    return y, hn


# speedup vs baseline: 47.3543x; 1.0471x over previous
import functools

import jax
import jax.numpy as jnp
import numpy as np
from jax import lax
from jax.experimental import pallas as pl
from jax.experimental.pallas import tpu as pltpu
from jax.experimental.pallas import tpu_sc as plsc

_EPS = 1e-5
_N = 1024
_B = 16
_R = _B * _N
_NW = 32
_INV2SIG2 = 12.5


def _mm(a, b):
    return jax.lax.dot_general(a, b, (((1,), (0,)), ((), ())),
                               preferred_element_type=jnp.float32)


def _relu(v):
    return jnp.maximum(v, 0.0)


def _make_gather_max(D, cs):
    per_w = _R // _NW
    n_sub = per_w // cs
    mesh = plsc.VectorSubcoreMesh(core_axis_name="c", subcore_axis_name="s", num_cores=2, num_subcores=16)

    @functools.partial(
        pl.kernel, mesh=mesh,
        out_type=jax.ShapeDtypeStruct((_R, D), jnp.float32),
        scratch_types=[
            pltpu.VMEM((cs,), jnp.int32),
            pltpu.VMEM((cs,), jnp.int32),
            pltpu.VMEM((cs,), jnp.int32),
            pltpu.VMEM((cs, D), jnp.float32),
            pltpu.VMEM((cs, D), jnp.float32),
            pltpu.VMEM((cs, D), jnp.float32),
            pltpu.VMEM((cs, D), jnp.float32),
            pltpu.SemaphoreType.DMA,
        ],
    )
    def gmax(tab, i0, i1, i2, out, i0_v, i1_v, i2_v, s_v, g0_v, g1_v, g2_v, sem):
        wid = lax.axis_index("s") * 2 + lax.axis_index("c")
        for j in range(n_sub):
            base = wid * per_w + j * cs
            pltpu.sync_copy(i0.at[pl.ds(base, cs)], i0_v)
            pltpu.sync_copy(i1.at[pl.ds(base, cs)], i1_v)
            pltpu.sync_copy(i2.at[pl.ds(base, cs)], i2_v)
            cps = [pltpu.async_copy(tab.at[pl.ds(base, cs)], s_v, sem),
                   pltpu.async_copy(tab.at[i0_v], g0_v, sem),
                   pltpu.async_copy(tab.at[i1_v], g1_v, sem),
                   pltpu.async_copy(tab.at[i2_v], g2_v, sem)]
            for cp in cps:
                cp.wait()

            def row(r, carry):
                for c in range(D // 16):
                    sl = pl.ds(c * 16, 16)
                    m = jnp.maximum(jnp.maximum(s_v[r, sl], g0_v[r, sl]),
                                    jnp.maximum(g1_v[r, sl], g2_v[r, sl]))
                    s_v[r, sl] = m
                return carry

            lax.fori_loop(0, cs, row, 0)
            pltpu.sync_copy(s_v, out.at[pl.ds(base, cs)])

    return gmax


def _make_gather3(D, cs):
    per_w = _R // _NW
    n_sub = per_w // cs
    mesh = plsc.VectorSubcoreMesh(core_axis_name="c", subcore_axis_name="s", num_cores=2, num_subcores=16)
    ot = jax.ShapeDtypeStruct((_R, D), jnp.float32)

    @functools.partial(
        pl.kernel, mesh=mesh,
        out_type=[ot, ot, ot],
        scratch_types=[
            pltpu.VMEM((cs,), jnp.int32),
            pltpu.VMEM((cs,), jnp.int32),
            pltpu.VMEM((cs,), jnp.int32),
            pltpu.VMEM((cs, D), jnp.float32),
            pltpu.VMEM((cs, D), jnp.float32),
            pltpu.VMEM((cs, D), jnp.float32),
            pltpu.SemaphoreType.DMA,
        ],
    )
    def g3(tab, i0, i1, i2, o0, o1, o2, i0_v, i1_v, i2_v, g0_v, g1_v, g2_v, sem):
        wid = lax.axis_index("s") * 2 + lax.axis_index("c")
        for j in range(n_sub):
            base = wid * per_w + j * cs
            pltpu.sync_copy(i0.at[pl.ds(base, cs)], i0_v)
            pltpu.sync_copy(i1.at[pl.ds(base, cs)], i1_v)
            pltpu.sync_copy(i2.at[pl.ds(base, cs)], i2_v)
            cps = [pltpu.async_copy(tab.at[i0_v], g0_v, sem),
                   pltpu.async_copy(tab.at[i1_v], g1_v, sem),
                   pltpu.async_copy(tab.at[i2_v], g2_v, sem)]
            for cp in cps:
                cp.wait()
            pltpu.sync_copy(g0_v, o0.at[pl.ds(base, cs)])
            pltpu.sync_copy(g1_v, o1.at[pl.ds(base, cs)])
            pltpu.sync_copy(g2_v, o2.at[pl.ds(base, cs)])

    return g3


def _front_body(face_ref, gn0_ref, gn1_ref, gn2_ref,
                sp1W, sp1b, sp2W, sp2b,
                r1W0, r1W1, r1W2, r1b, r2W, r2b,
                f1W, f1b, f2W, f2b,
                Wm, wsq, fks, fkb,
                s1Wa, s1Wb, s1Wc, s1b, s2W, s2b,
                c1Wa, c1Wb, c1b,
                st0_ref, sp1_ref):
    f = face_ref[0]
    centers = f[:, 0:3]
    corners = f[:, 3:12] - jnp.concatenate([centers, centers, centers], axis=1)
    nrm = f[:, 12:15]

    sf0 = _relu(_mm(_relu(_mm(centers, sp1W[...]) + sp1b[...]), sp2W[...]) + sp2b[...])

    def rmlp(W1):
        z = _relu(_mm(corners, W1[...]) + r1b[...])
        return _relu(_mm(z, r2W[...]) + r2b[...])
    fea = (rmlp(r1W0) + rmlp(r1W1) + rmlp(r1W2)) * (1.0 / 3.0)
    frc = _relu(_mm(_relu(_mm(fea, f1W[...]) + f1b[...]), f2W[...]) + f2b[...])

    acc = jnp.zeros((_N, 256), jnp.float32)
    for F in (nrm, gn0_ref[:, 0:3], gn1_ref[:, 0:3], gn2_ref[:, 0:3]):
        fsq = jnp.sum(F * F, axis=1, keepdims=True)
        d = fsq + wsq[...] - 2.0 * _mm(F, Wm[...])
        acc = acc + jnp.exp(-_INV2SIG2 * d)
    fkc = (acc[:, 0:64] + acc[:, 64:128] + acc[:, 128:192] + acc[:, 192:256]) * (1.0 / 16.0)
    fkc = _relu(fkc * fks[...] + fkb[...])

    st = _relu(_mm(frc, s1Wa[...]) + _mm(fkc, s1Wb[...]) + _mm(nrm, s1Wc[...]) + s1b[...])
    st0 = _relu(_mm(st, s2W[...]) + s2b[...])
    st0_ref[...] = st0
    sp1_ref[...] = _relu(_mm(sf0, c1Wa[...]) + _mm(st0, c1Wb[...]) + c1b[...])


def _mid_body(gm0_ref, sp1_ref, a1W, a1b, c2Wa, c2Wb, c2b, st1_ref, sp2_ref):
    st1 = _relu(_mm(gm0_ref[...], a1W[...]) + a1b[...])
    st1_ref[...] = st1
    sp2_ref[...] = _relu(_mm(sp1_ref[...], c2Wa[...]) + _mm(st1, c2Wb[...]) + c2b[...])


def _tail_body(gm1_ref, sp1_ref, sp2_ref,
               a2W, a2b, fuWa, fuWb, fub, caWa, caWb, caWc, cab, out_ref):
    sp2 = sp2_ref[...]
    st2 = _relu(_mm(gm1_ref[...], a2W[...]) + a2b[...])
    sp3 = _relu(_mm(sp2, fuWa[...]) + _mm(st2, fuWb[...]) + fub[...])
    fea2 = _relu(_mm(sp1_ref[...], caWa[...]) + _mm(sp2, caWb[...]) +
                 _mm(sp3, caWc[...]) + cab[...])
    out_ref[0] = jnp.max(fea2, axis=0, keepdims=True)


def _head_body(fm_ref, W1, b1, W2, b2, W3, b3, y_ref, hn_ref):
    h = _relu(_mm(fm_ref[...], W1[...]) + b1[...])
    h = _relu(_mm(h, W2[...]) + b2[...])
    y_ref[...] = _mm(h, W3[...]) + b3[...]
    hn_ref[...] = h * jax.lax.rsqrt(jnp.sum(h * h))


def _fold(p, name):
    s = p[name + '_g'] * (1.0 / np.sqrt(1.0 + _EPS))
    W = (p[name + '_W'] * s[:, None]).T
    b = (p[name + '_b'] * s + p[name + '_be'])[None, :]
    return W, b


def _wspec(w):
    return pl.BlockSpec(w.shape, functools.partial(lambda b, _s: (0,) * len(_s), _s=w.shape))


@jax.jit
def kernel(x, params):
    p = params
    face = x[..., :15]
    nbr = x[..., 15:18].astype(jnp.int32)
    gidx = (nbr + (jnp.arange(_B, dtype=jnp.int32) * _N)[:, None, None]).reshape(_R, 3)
    gi0, gi1, gi2 = gidx[:, 0], gidx[:, 1], gidx[:, 2]

    nrm_tab = jnp.pad(face[..., 12:15].reshape(_R, 3), ((0, 0), (0, 125)))

    sp1W, sp1b = _fold(p, 'sp1')
    sp2W, sp2b = _fold(p, 'sp2')
    r1W, r1b = _fold(p, 'rot1')
    z3 = jnp.zeros((3, 32), jnp.float32)
    r1W0 = jnp.concatenate([r1W, z3], axis=0)
    r1W1 = jnp.concatenate([z3, r1W], axis=0)
    r1W2 = jnp.concatenate([r1W[3:6], z3, r1W[0:3]], axis=0)
    r2W, r2b = _fold(p, 'rot2')
    f1W, f1b = _fold(p, 'fus1')
    f2W, f2b = _fold(p, 'fus2')

    alpha, beta = p['fkc_alpha'], p['fkc_beta']
    w3 = jnp.concatenate([jnp.sin(alpha) * jnp.cos(beta),
                          jnp.sin(alpha) * jnp.sin(beta),
                          jnp.cos(alpha)], axis=0)
    Wm = jnp.transpose(w3, (0, 2, 1)).reshape(3, 256)
    wsq = jnp.sum(Wm * Wm, axis=0)[None, :]
    fks = (p['fkc_g'] * (1.0 / np.sqrt(1.0 + _EPS)))[None, :]
    fkb = p['fkc_be'][None, :]

    s1W, s1b = _fold(p, 'str1')
    s1Wa, s1Wb, s1Wc = s1W[0:64], s1W[64:128], s1W[128:131]
    s2W, s2b = _fold(p, 'str2')
    s2W = jnp.pad(s2W, ((0, 0), (0, 125)))
    s2b = jnp.pad(s2b, ((0, 0), (0, 125)))
    c1W, c1b = _fold(p, 'mc1_comb')
    c1Wa = c1W[0:64]
    c1Wb = jnp.pad(c1W[64:195], ((0, 125), (0, 0)))
    a1W, a1b = _fold(p, 'mc1_agg')
    a1W = jnp.pad(a1W, ((0, 125), (0, 0)))
    c2W, c2b = _fold(p, 'mc2_comb')
    c2Wa, c2Wb = c2W[0:256], c2W[256:512]
    a2W, a2b = _fold(p, 'mc2_agg')
    fuW, fub = _fold(p, 'fusion')
    fuWa, fuWb = fuW[0:512], fuW[512:1024]
    caW, cab = _fold(p, 'cat')
    caWa, caWb, caWc = caW[0:256], caW[256:768], caW[768:1792]

    g0, g1, g2 = _make_gather3(128, 128)(nrm_tab, gi0, gi1, gi2)

    fw = [sp1W, sp1b, sp2W, sp2b,
          r1W0, r1W1, r1W2, r1b, r2W, r2b,
          f1W, f1b, f2W, f2b,
          Wm, wsq, fks, fkb,
          s1Wa, s1Wb, s1Wc, s1b, s2W, s2b,
          c1Wa, c1Wb, c1b]
    st0_tab, sp1_tab = pl.pallas_call(
        _front_body,
        grid=(_B,),
        in_specs=[pl.BlockSpec((1, _N, 15), lambda b: (b, 0, 0)),
                  pl.BlockSpec((_N, 128), lambda b: (b, 0)),
                  pl.BlockSpec((_N, 128), lambda b: (b, 0)),
                  pl.BlockSpec((_N, 128), lambda b: (b, 0))] + [_wspec(w) for w in fw],
        out_specs=[pl.BlockSpec((_N, 256), lambda b: (b, 0)),
                   pl.BlockSpec((_N, 256), lambda b: (b, 0))],
        out_shape=[jax.ShapeDtypeStruct((_R, 256), jnp.float32),
                   jax.ShapeDtypeStruct((_R, 256), jnp.float32)],
        compiler_params=pltpu.CompilerParams(dimension_semantics=("arbitrary",)),
    )(face, g0, g1, g2, *fw)

    gm0 = _make_gather_max(256, 64)(st0_tab, gi0, gi1, gi2)

    mw = [a1W, a1b, c2Wa, c2Wb, c2b]
    st1_tab, sp2_tab = pl.pallas_call(
        _mid_body,
        grid=(_B,),
        in_specs=[pl.BlockSpec((_N, 256), lambda b: (b, 0)),
                  pl.BlockSpec((_N, 256), lambda b: (b, 0))] + [_wspec(w) for w in mw],
        out_specs=[pl.BlockSpec((_N, 256), lambda b: (b, 0)),
                   pl.BlockSpec((_N, 512), lambda b: (b, 0))],
        out_shape=[jax.ShapeDtypeStruct((_R, 256), jnp.float32),
                   jax.ShapeDtypeStruct((_R, 512), jnp.float32)],
        compiler_params=pltpu.CompilerParams(dimension_semantics=("arbitrary",)),
    )(gm0, sp1_tab, *mw)

    gm1 = _make_gather_max(256, 64)(st1_tab, gi0, gi1, gi2)

    tw = [a2W, a2b, fuWa, fuWb, fub, caWa, caWb, caWc, cab]
    fm = pl.pallas_call(
        _tail_body,
        grid=(_B,),
        in_specs=[pl.BlockSpec((_N, 256), lambda b: (b, 0)),
                  pl.BlockSpec((_N, 256), lambda b: (b, 0)),
                  pl.BlockSpec((_N, 512), lambda b: (b, 0))] + [_wspec(w) for w in tw],
        out_specs=pl.BlockSpec((1, 1, 1024), lambda b: (b, 0, 0)),
        out_shape=jax.ShapeDtypeStruct((_B, 1, 1024), jnp.float32),
        compiler_params=pltpu.CompilerParams(dimension_semantics=("arbitrary",)),
    )(gm1, sp1_tab, sp2_tab, *tw)
    fm = fm.reshape(_B, 1024)

    hw = [p['l1_W'].T, p['l1_b'][None, :],
          p['l2_W'].T, p['l2_b'][None, :],
          p['l3_W'].T, p['l3_b'][None, :]]
    y, hn = pl.pallas_call(
        _head_body,
        in_specs=[pl.BlockSpec(fm.shape, lambda: (0, 0))] +
                 [pl.BlockSpec(w.shape, lambda: (0, 0)) for w in hw],
        out_specs=[pl.BlockSpec((_B, 40), lambda: (0, 0)),
                   pl.BlockSpec((_B, 256), lambda: (0, 0))],
        out_shape=[jax.ShapeDtypeStruct((_B, 40), jnp.float32),
                   jax.ShapeDtypeStruct((_B, 256), jnp.float32)],
    )(fm, *hw)
    return y, hn


# speedup vs baseline: 48.3351x; 1.0207x over previous
import functools

import jax
import jax.numpy as jnp
import numpy as np
from jax import lax
from jax.experimental import pallas as pl
from jax.experimental.pallas import tpu as pltpu
from jax.experimental.pallas import tpu_sc as plsc

_EPS = 1e-5
_N = 1024
_B = 16
_R = _B * _N
_NW = 32
_INV2SIG2 = 12.5


def _mm(a, b):
    return jax.lax.dot_general(a, b, (((1,), (0,)), ((), ())),
                               preferred_element_type=jnp.float32)


def _relu(v):
    return jnp.maximum(v, 0.0)


_CS = 32


def _make_gather_max3(D):
    per_w = _R // _NW
    n_sub = per_w // _CS
    mesh = plsc.VectorSubcoreMesh(core_axis_name="c", subcore_axis_name="s",
                                  num_cores=2, num_subcores=16)
    row_bufs = [pltpu.VMEM((_CS, D), jnp.float32) for _ in range(6)]

    @functools.partial(
        pl.kernel, mesh=mesh,
        out_type=jax.ShapeDtypeStruct((_R, D), jnp.float32),
        scratch_types=[
            pltpu.VMEM((n_sub, _CS), jnp.int32),
            pltpu.VMEM((n_sub, _CS), jnp.int32),
            pltpu.VMEM((n_sub, _CS), jnp.int32),
        ] + row_bufs + [pltpu.SemaphoreType.DMA, pltpu.SemaphoreType.DMA],
    )
    def gmax(tab, i0, i1, i2, out,
             iv0, iv1, iv2, ga0, ga1, ga2, gb0, gb1, gb2, sa, sb):
        wid = lax.axis_index("s") * 2 + lax.axis_index("c")
        pltpu.sync_copy(i0.at[pl.ds(wid * n_sub, n_sub)], iv0)
        pltpu.sync_copy(i1.at[pl.ds(wid * n_sub, n_sub)], iv1)
        pltpu.sync_copy(i2.at[pl.ds(wid * n_sub, n_sub)], iv2)
        stages = [((ga0, ga1, ga2), sa), ((gb0, gb1, gb2), sb)]

        def issue(j, stage):
            bufs, sem = stage
            return [pltpu.async_copy(tab.at[iv.at[j]], buf, sem)
                    for iv, buf in zip((iv0, iv1, iv2), bufs)]

        def crunch(j, stage, cps):
            (b0, b1, b2), _ = stage
            for cp in cps:
                cp.wait()

            def row(r, carry):
                for c in range(D // 16):
                    sl = pl.ds(c * 16, 16)
                    b0[r, sl] = jnp.maximum(jnp.maximum(b0[r, sl], b1[r, sl]),
                                            b2[r, sl])
                return carry

            lax.fori_loop(0, _CS, row, 0)
            pltpu.sync_copy(b0, out.at[pl.ds(wid * per_w + j * _CS, _CS)])

        pend = issue(0, stages[0])
        for j in range(1, n_sub):
            nxt = issue(j, stages[j % 2])
            crunch(j - 1, stages[(j - 1) % 2], pend)
            pend = nxt
        crunch(n_sub - 1, stages[(n_sub - 1) % 2], pend)

    return gmax


def _gather(P, M):
    hi = M.astype(jnp.bfloat16)
    lo = (M - hi.astype(jnp.float32)).astype(jnp.bfloat16)
    return _mm(P, hi) + _mm(P, lo)


def _front_body(face_ref, nbr_ref,
                sp1W, sp1b, sp2W, sp2b,
                r1W0, r1W1, r1W2, r1b, r2W, r2b,
                f1W, f1b, f2W, f2b,
                Wm, wsq, fks, fkb,
                s1Wa, s1Wb, s1Wc, s1b, s2W, s2b,
                c1Wa, c1Wb, c1b,
                st0_ref, sp1_ref):
    f = face_ref[0]
    centers = f[:, 0:3]
    corners = f[:, 3:12] - jnp.concatenate([centers, centers, centers], axis=1)
    nrm = f[:, 12:15]

    sf0 = _relu(_mm(_relu(_mm(centers, sp1W[...]) + sp1b[...]), sp2W[...]) + sp2b[...])

    def rmlp(W1):
        z = _relu(_mm(corners, W1[...]) + r1b[...])
        return _relu(_mm(z, r2W[...]) + r2b[...])
    fea = (rmlp(r1W0) + rmlp(r1W1) + rmlp(r1W2)) * (1.0 / 3.0)
    frc = _relu(_mm(_relu(_mm(fea, f1W[...]) + f1b[...]), f2W[...]) + f2b[...])

    nbr = nbr_ref[0]
    lane = jax.lax.broadcasted_iota(jnp.int32, (_N, _N), 1)
    acc = jnp.zeros((_N, 256), jnp.float32)
    for F in (nrm,
              _gather((lane == nbr[:, 0:1]).astype(jnp.bfloat16), nrm),
              _gather((lane == nbr[:, 1:2]).astype(jnp.bfloat16), nrm),
              _gather((lane == nbr[:, 2:3]).astype(jnp.bfloat16), nrm)):
        fsq = jnp.sum(F * F, axis=1, keepdims=True)
        d = fsq + wsq[...] - 2.0 * _mm(F, Wm[...])
        acc = acc + jnp.exp(-_INV2SIG2 * d)
    fkc = (acc[:, 0:64] + acc[:, 64:128] + acc[:, 128:192] + acc[:, 192:256]) * (1.0 / 16.0)
    fkc = _relu(fkc * fks[...] + fkb[...])

    st = _relu(_mm(frc, s1Wa[...]) + _mm(fkc, s1Wb[...]) + _mm(nrm, s1Wc[...]) + s1b[...])
    st0 = _relu(_mm(st, s2W[...]) + s2b[...])
    st0_ref[...] = st0
    sp1_ref[...] = _relu(_mm(sf0, c1Wa[...]) + _mm(st0, c1Wb[...]) + c1b[...])


def _mid_body(gm0_ref, st0_ref, sp1_ref, a1W, a1b, c2Wa, c2Wb, c2b, st1_ref, sp2_ref):
    st1 = _relu(_mm(jnp.maximum(gm0_ref[...], st0_ref[...]), a1W[...]) + a1b[...])
    st1_ref[...] = st1
    sp2_ref[...] = _relu(_mm(sp1_ref[...], c2Wa[...]) + _mm(st1, c2Wb[...]) + c2b[...])


def _tail_body(gm1_ref, st1_ref, sp1_ref, sp2_ref,
               a2W, a2b, fuWa, fuWb, fub, caWa, caWb, caWc, cab, out_ref):
    sp2 = sp2_ref[...]
    st2 = _relu(_mm(jnp.maximum(gm1_ref[...], st1_ref[...]), a2W[...]) + a2b[...])
    sp3 = _relu(_mm(sp2, fuWa[...]) + _mm(st2, fuWb[...]) + fub[...])
    fea2 = _relu(_mm(sp1_ref[...], caWa[...]) + _mm(sp2, caWb[...]) +
                 _mm(sp3, caWc[...]) + cab[...])
    out_ref[0] = jnp.max(fea2, axis=0, keepdims=True)


def _head_body(fm_ref, W1, b1, W2, b2, W3, b3, y_ref, hn_ref):
    h = _relu(_mm(fm_ref[...], W1[...]) + b1[...])
    h = _relu(_mm(h, W2[...]) + b2[...])
    y_ref[...] = _mm(h, W3[...]) + b3[...]
    hn_ref[...] = h * jax.lax.rsqrt(jnp.sum(h * h))


def _fold(p, name):
    s = p[name + '_g'] * (1.0 / np.sqrt(1.0 + _EPS))
    W = (p[name + '_W'] * s[:, None]).T
    b = (p[name + '_b'] * s + p[name + '_be'])[None, :]
    return W, b


def _wspec(w):
    return pl.BlockSpec(w.shape, functools.partial(lambda b, _s: (0,) * len(_s), _s=w.shape))


@jax.jit
def kernel(x, params):
    p = params
    face = x[..., :15]
    nbr = x[..., 15:18].astype(jnp.int32)
    gidx = (nbr + (jnp.arange(_B, dtype=jnp.int32) * _N)[:, None, None]).reshape(_R, 3)
    gi0 = gidx[:, 0].reshape(_R // _CS, _CS)
    gi1 = gidx[:, 1].reshape(_R // _CS, _CS)
    gi2 = gidx[:, 2].reshape(_R // _CS, _CS)

    sp1W, sp1b = _fold(p, 'sp1')
    sp2W, sp2b = _fold(p, 'sp2')
    r1W, r1b = _fold(p, 'rot1')
    z3 = jnp.zeros((3, 32), jnp.float32)
    r1W0 = jnp.concatenate([r1W, z3], axis=0)
    r1W1 = jnp.concatenate([z3, r1W], axis=0)
    r1W2 = jnp.concatenate([r1W[3:6], z3, r1W[0:3]], axis=0)
    r2W, r2b = _fold(p, 'rot2')
    f1W, f1b = _fold(p, 'fus1')
    f2W, f2b = _fold(p, 'fus2')

    alpha, beta = p['fkc_alpha'], p['fkc_beta']
    w3 = jnp.concatenate([jnp.sin(alpha) * jnp.cos(beta),
                          jnp.sin(alpha) * jnp.sin(beta),
                          jnp.cos(alpha)], axis=0)
    Wm = jnp.transpose(w3, (0, 2, 1)).reshape(3, 256)
    wsq = jnp.sum(Wm * Wm, axis=0)[None, :]
    fks = (p['fkc_g'] * (1.0 / np.sqrt(1.0 + _EPS)))[None, :]
    fkb = p['fkc_be'][None, :]

    s1W, s1b = _fold(p, 'str1')
    s1Wa, s1Wb, s1Wc = s1W[0:64], s1W[64:128], s1W[128:131]
    s2W, s2b = _fold(p, 'str2')
    s2W = jnp.pad(s2W, ((0, 0), (0, 125)))
    s2b = jnp.pad(s2b, ((0, 0), (0, 125)))
    c1W, c1b = _fold(p, 'mc1_comb')
    c1Wa = c1W[0:64]
    c1Wb = jnp.pad(c1W[64:195], ((0, 125), (0, 0)))
    a1W, a1b = _fold(p, 'mc1_agg')
    a1W = jnp.pad(a1W, ((0, 125), (0, 0)))
    c2W, c2b = _fold(p, 'mc2_comb')
    c2Wa, c2Wb = c2W[0:256], c2W[256:512]
    a2W, a2b = _fold(p, 'mc2_agg')
    fuW, fub = _fold(p, 'fusion')
    fuWa, fuWb = fuW[0:512], fuW[512:1024]
    caW, cab = _fold(p, 'cat')
    caWa, caWb, caWc = caW[0:256], caW[256:768], caW[768:1792]

    fw = [sp1W, sp1b, sp2W, sp2b,
          r1W0, r1W1, r1W2, r1b, r2W, r2b,
          f1W, f1b, f2W, f2b,
          Wm, wsq, fks, fkb,
          s1Wa, s1Wb, s1Wc, s1b, s2W, s2b,
          c1Wa, c1Wb, c1b]
    st0_tab, sp1_tab = pl.pallas_call(
        _front_body,
        grid=(_B,),
        in_specs=[pl.BlockSpec((1, _N, 15), lambda b: (b, 0, 0)),
                  pl.BlockSpec((1, _N, 3), lambda b: (b, 0, 0))] + [_wspec(w) for w in fw],
        out_specs=[pl.BlockSpec((_N, 256), lambda b: (b, 0)),
                   pl.BlockSpec((_N, 256), lambda b: (b, 0))],
        out_shape=[jax.ShapeDtypeStruct((_R, 256), jnp.float32),
                   jax.ShapeDtypeStruct((_R, 256), jnp.float32)],
        compiler_params=pltpu.CompilerParams(dimension_semantics=("arbitrary",)),
    )(face, nbr, *fw)

    _gmax = _make_gather_max3(256)
    gm0 = _gmax(st0_tab, gi0, gi1, gi2)

    mw = [a1W, a1b, c2Wa, c2Wb, c2b]
    st1_tab, sp2_tab = pl.pallas_call(
        _mid_body,
        grid=(_B,),
        in_specs=[pl.BlockSpec((_N, 256), lambda b: (b, 0)),
                  pl.BlockSpec((_N, 256), lambda b: (b, 0)),
                  pl.BlockSpec((_N, 256), lambda b: (b, 0))] + [_wspec(w) for w in mw],
        out_specs=[pl.BlockSpec((_N, 256), lambda b: (b, 0)),
                   pl.BlockSpec((_N, 512), lambda b: (b, 0))],
        out_shape=[jax.ShapeDtypeStruct((_R, 256), jnp.float32),
                   jax.ShapeDtypeStruct((_R, 512), jnp.float32)],
        compiler_params=pltpu.CompilerParams(dimension_semantics=("arbitrary",)),
    )(gm0, st0_tab, sp1_tab, *mw)

    gm1 = _gmax(st1_tab, gi0, gi1, gi2)

    tw = [a2W, a2b, fuWa, fuWb, fub, caWa, caWb, caWc, cab]
    fm = pl.pallas_call(
        _tail_body,
        grid=(_B,),
        in_specs=[pl.BlockSpec((_N, 256), lambda b: (b, 0)),
                  pl.BlockSpec((_N, 256), lambda b: (b, 0)),
                  pl.BlockSpec((_N, 256), lambda b: (b, 0)),
                  pl.BlockSpec((_N, 512), lambda b: (b, 0))] + [_wspec(w) for w in tw],
        out_specs=pl.BlockSpec((1, 1, 1024), lambda b: (b, 0, 0)),
        out_shape=jax.ShapeDtypeStruct((_B, 1, 1024), jnp.float32),
        compiler_params=pltpu.CompilerParams(dimension_semantics=("arbitrary",)),
    )(gm1, st1_tab, sp1_tab, sp2_tab, *tw)
    fm = fm.reshape(_B, 1024)

    hw = [p['l1_W'].T, p['l1_b'][None, :],
          p['l2_W'].T, p['l2_b'][None, :],
          p['l3_W'].T, p['l3_b'][None, :]]
    y, hn = pl.pallas_call(
        _head_body,
        in_specs=[pl.BlockSpec(fm.shape, lambda: (0, 0))] +
                 [pl.BlockSpec(w.shape, lambda: (0, 0)) for w in hw],
        out_specs=[pl.BlockSpec((_B, 40), lambda: (0, 0)),
                   pl.BlockSpec((_B, 256), lambda: (0, 0))],
        out_shape=[jax.ShapeDtypeStruct((_B, 40), jnp.float32),
                   jax.ShapeDtypeStruct((_B, 256), jnp.float32)],
    )(fm, *hw)
    return y, hn


# speedup vs baseline: 48.4329x; 1.0020x over previous
import functools

import jax
import jax.numpy as jnp
import numpy as np
from jax import lax
from jax.experimental import pallas as pl
from jax.experimental.pallas import tpu as pltpu
from jax.experimental.pallas import tpu_sc as plsc

_EPS = 1e-5
_N = 1024
_B = 16
_R = _B * _N
_NW = 32
_INV2SIG2 = 12.5


def _mm(a, b):
    return jax.lax.dot_general(a, b, (((1,), (0,)), ((), ())),
                               preferred_element_type=jnp.float32)


def _relu(v):
    return jnp.maximum(v, 0.0)


_CS = 32


def _make_gather_max3(D, R):
    per_w = R // _NW
    n_sub = per_w // _CS
    mesh = plsc.VectorSubcoreMesh(core_axis_name="c", subcore_axis_name="s",
                                  num_cores=2, num_subcores=16)
    row_bufs = [pltpu.VMEM((_CS, D), jnp.float32) for _ in range(6)]

    @functools.partial(
        pl.kernel, mesh=mesh,
        out_type=jax.ShapeDtypeStruct((R, D), jnp.float32),
        scratch_types=[
            pltpu.VMEM((n_sub, _CS), jnp.int32),
            pltpu.VMEM((n_sub, _CS), jnp.int32),
            pltpu.VMEM((n_sub, _CS), jnp.int32),
        ] + row_bufs + [pltpu.SemaphoreType.DMA, pltpu.SemaphoreType.DMA],
    )
    def gmax(tab, i0, i1, i2, out,
             iv0, iv1, iv2, ga0, ga1, ga2, gb0, gb1, gb2, sa, sb):
        wid = lax.axis_index("s") * 2 + lax.axis_index("c")
        pltpu.sync_copy(i0.at[pl.ds(wid * n_sub, n_sub)], iv0)
        pltpu.sync_copy(i1.at[pl.ds(wid * n_sub, n_sub)], iv1)
        pltpu.sync_copy(i2.at[pl.ds(wid * n_sub, n_sub)], iv2)
        stages = [((ga0, ga1, ga2), sa), ((gb0, gb1, gb2), sb)]

        def issue(j, stage):
            bufs, sem = stage
            return [pltpu.async_copy(tab.at[iv.at[j]], buf, sem)
                    for iv, buf in zip((iv0, iv1, iv2), bufs)]

        def crunch(j, stage, cps):
            (b0, b1, b2), _ = stage
            for cp in cps:
                cp.wait()

            def row(r, carry):
                for c in range(D // 16):
                    sl = pl.ds(c * 16, 16)
                    b0[r, sl] = jnp.maximum(jnp.maximum(b0[r, sl], b1[r, sl]),
                                            b2[r, sl])
                return carry

            lax.fori_loop(0, _CS, row, 0)
            pltpu.sync_copy(b0, out.at[pl.ds(wid * per_w + j * _CS, _CS)])

        pend = issue(0, stages[0])
        for j in range(1, n_sub):
            nxt = issue(j, stages[j % 2])
            crunch(j - 1, stages[(j - 1) % 2], pend)
            pend = nxt
        crunch(n_sub - 1, stages[(n_sub - 1) % 2], pend)

    return gmax


def _gather(P, M):
    hi = M.astype(jnp.bfloat16)
    lo = (M - hi.astype(jnp.float32)).astype(jnp.bfloat16)
    return _mm(P, hi) + _mm(P, lo)


def _front_body(face_ref, nbr_ref,
                sp1W, sp1b, sp2W, sp2b,
                r1W0, r1W1, r1W2, r1b, r2W, r2b,
                f1W, f1b, f2W, f2b,
                Wm, wsq, fks, fkb,
                s1Wa, s1Wb, s1Wc, s1b, s2W, s2b,
                c1Wa, c1Wb, c1b,
                st0_ref, sp1_ref):
    f = face_ref[0]
    centers = f[:, 0:3]
    corners = f[:, 3:12] - jnp.concatenate([centers, centers, centers], axis=1)
    nrm = f[:, 12:15]

    sf0 = _relu(_mm(_relu(_mm(centers, sp1W[...]) + sp1b[...]), sp2W[...]) + sp2b[...])

    def rmlp(W1):
        z = _relu(_mm(corners, W1[...]) + r1b[...])
        return _relu(_mm(z, r2W[...]) + r2b[...])
    fea = (rmlp(r1W0) + rmlp(r1W1) + rmlp(r1W2)) * (1.0 / 3.0)
    frc = _relu(_mm(_relu(_mm(fea, f1W[...]) + f1b[...]), f2W[...]) + f2b[...])

    nbr = nbr_ref[0]
    lane = jax.lax.broadcasted_iota(jnp.int32, (_N, _N), 1)
    acc = jnp.zeros((_N, 256), jnp.float32)
    for F in (nrm,
              _gather((lane == nbr[:, 0:1]).astype(jnp.bfloat16), nrm),
              _gather((lane == nbr[:, 1:2]).astype(jnp.bfloat16), nrm),
              _gather((lane == nbr[:, 2:3]).astype(jnp.bfloat16), nrm)):
        fsq = jnp.sum(F * F, axis=1, keepdims=True)
        d = fsq + wsq[...] - 2.0 * _mm(F, Wm[...])
        acc = acc + jnp.exp(-_INV2SIG2 * d)
    fkc = (acc[:, 0:64] + acc[:, 64:128] + acc[:, 128:192] + acc[:, 192:256]) * (1.0 / 16.0)
    fkc = _relu(fkc * fks[...] + fkb[...])

    st = _relu(_mm(frc, s1Wa[...]) + _mm(fkc, s1Wb[...]) + _mm(nrm, s1Wc[...]) + s1b[...])
    st0 = _relu(_mm(st, s2W[...]) + s2b[...])
    st0_ref[...] = st0
    sp1_ref[...] = _relu(_mm(sf0, c1Wa[...]) + _mm(st0, c1Wb[...]) + c1b[...])


def _mid_body(gm0_ref, st0_ref, sp1_ref, a1W, a1b, c2Wa, c2Wb, c2b, st1_ref, sp2_ref):
    st1 = _relu(_mm(jnp.maximum(gm0_ref[...], st0_ref[...]), a1W[...]) + a1b[...])
    st1_ref[...] = st1
    sp2_ref[...] = _relu(_mm(sp1_ref[...], c2Wa[...]) + _mm(st1, c2Wb[...]) + c2b[...])


def _tail_body(gm1_ref, st1_ref, sp1_ref, sp2_ref,
               a2W, a2b, fuWa, fuWb, fub, caWa, caWb, caWc, cab, out_ref):
    sp2 = sp2_ref[...]
    st2 = _relu(_mm(jnp.maximum(gm1_ref[...], st1_ref[...]), a2W[...]) + a2b[...])
    sp3 = _relu(_mm(sp2, fuWa[...]) + _mm(st2, fuWb[...]) + fub[...])
    fea2 = _relu(_mm(sp1_ref[...], caWa[...]) + _mm(sp2, caWb[...]) +
                 _mm(sp3, caWc[...]) + cab[...])
    out_ref[0] = jnp.max(fea2, axis=0, keepdims=True)


def _head_body(fm_ref, W1, b1, W2, b2, W3, b3, y_ref, hn_ref):
    h = _relu(_mm(fm_ref[...], W1[...]) + b1[...])
    h = _relu(_mm(h, W2[...]) + b2[...])
    y_ref[...] = _mm(h, W3[...]) + b3[...]
    hn_ref[...] = h * jax.lax.rsqrt(jnp.sum(h * h))


def _fold(p, name):
    s = p[name + '_g'] * (1.0 / np.sqrt(1.0 + _EPS))
    W = (p[name + '_W'] * s[:, None]).T
    b = (p[name + '_b'] * s + p[name + '_be'])[None, :]
    return W, b


def _wspec(w):
    return pl.BlockSpec(w.shape, functools.partial(lambda b, _s: (0,) * len(_s), _s=w.shape))


_HB = _B // 2
_HR = _HB * _N


@jax.jit
def kernel(x, params):
    p = params
    face = x[..., :15]
    nbr = x[..., 15:18].astype(jnp.int32)
    gih = []
    for h in range(2):
        gidx = (nbr[h * _HB:(h + 1) * _HB] +
                (jnp.arange(_HB, dtype=jnp.int32) * _N)[:, None, None]).reshape(_HR, 3)
        gih.append(tuple(gidx[:, k].reshape(_HR // _CS, _CS) for k in range(3)))

    sp1W, sp1b = _fold(p, 'sp1')
    sp2W, sp2b = _fold(p, 'sp2')
    r1W, r1b = _fold(p, 'rot1')
    z3 = jnp.zeros((3, 32), jnp.float32)
    r1W0 = jnp.concatenate([r1W, z3], axis=0)
    r1W1 = jnp.concatenate([z3, r1W], axis=0)
    r1W2 = jnp.concatenate([r1W[3:6], z3, r1W[0:3]], axis=0)
    r2W, r2b = _fold(p, 'rot2')
    f1W, f1b = _fold(p, 'fus1')
    f2W, f2b = _fold(p, 'fus2')

    alpha, beta = p['fkc_alpha'], p['fkc_beta']
    w3 = jnp.concatenate([jnp.sin(alpha) * jnp.cos(beta),
                          jnp.sin(alpha) * jnp.sin(beta),
                          jnp.cos(alpha)], axis=0)
    Wm = jnp.transpose(w3, (0, 2, 1)).reshape(3, 256)
    wsq = jnp.sum(Wm * Wm, axis=0)[None, :]
    fks = (p['fkc_g'] * (1.0 / np.sqrt(1.0 + _EPS)))[None, :]
    fkb = p['fkc_be'][None, :]

    s1W, s1b = _fold(p, 'str1')
    s1Wa, s1Wb, s1Wc = s1W[0:64], s1W[64:128], s1W[128:131]
    s2W, s2b = _fold(p, 'str2')
    s2W = jnp.pad(s2W, ((0, 0), (0, 125)))
    s2b = jnp.pad(s2b, ((0, 0), (0, 125)))
    c1W, c1b = _fold(p, 'mc1_comb')
    c1Wa = c1W[0:64]
    c1Wb = jnp.pad(c1W[64:195], ((0, 125), (0, 0)))
    a1W, a1b = _fold(p, 'mc1_agg')
    a1W = jnp.pad(a1W, ((0, 125), (0, 0)))
    c2W, c2b = _fold(p, 'mc2_comb')
    c2Wa, c2Wb = c2W[0:256], c2W[256:512]
    a2W, a2b = _fold(p, 'mc2_agg')
    fuW, fub = _fold(p, 'fusion')
    fuWa, fuWb = fuW[0:512], fuW[512:1024]
    caW, cab = _fold(p, 'cat')
    caWa, caWb, caWc = caW[0:256], caW[256:768], caW[768:1792]

    fw = [sp1W, sp1b, sp2W, sp2b,
          r1W0, r1W1, r1W2, r1b, r2W, r2b,
          f1W, f1b, f2W, f2b,
          Wm, wsq, fks, fkb,
          s1Wa, s1Wb, s1Wc, s1b, s2W, s2b,
          c1Wa, c1Wb, c1b]
    mw = [a1W, a1b, c2Wa, c2Wb, c2b]
    tw = [a2W, a2b, fuWa, fuWb, fub, caWa, caWb, caWc, cab]

    def front(h):
        return pl.pallas_call(
            _front_body,
            grid=(_HB,),
            in_specs=[pl.BlockSpec((1, _N, 15), lambda b: (b, 0, 0)),
                      pl.BlockSpec((1, _N, 3), lambda b: (b, 0, 0))] + [_wspec(w) for w in fw],
            out_specs=[pl.BlockSpec((_N, 256), lambda b: (b, 0)),
                       pl.BlockSpec((_N, 256), lambda b: (b, 0))],
            out_shape=[jax.ShapeDtypeStruct((_HR, 256), jnp.float32),
                       jax.ShapeDtypeStruct((_HR, 256), jnp.float32)],
            compiler_params=pltpu.CompilerParams(dimension_semantics=("arbitrary",)),
        )(face[h * _HB:(h + 1) * _HB], nbr[h * _HB:(h + 1) * _HB], *fw)

    def mid(gm0, st0_tab, sp1_tab):
        return pl.pallas_call(
            _mid_body,
            grid=(_HB,),
            in_specs=[pl.BlockSpec((_N, 256), lambda b: (b, 0)),
                      pl.BlockSpec((_N, 256), lambda b: (b, 0)),
                      pl.BlockSpec((_N, 256), lambda b: (b, 0))] + [_wspec(w) for w in mw],
            out_specs=[pl.BlockSpec((_N, 256), lambda b: (b, 0)),
                       pl.BlockSpec((_N, 512), lambda b: (b, 0))],
            out_shape=[jax.ShapeDtypeStruct((_HR, 256), jnp.float32),
                       jax.ShapeDtypeStruct((_HR, 512), jnp.float32)],
            compiler_params=pltpu.CompilerParams(dimension_semantics=("arbitrary",)),
        )(gm0, st0_tab, sp1_tab, *mw)

    def tail(gm1, st1_tab, sp1_tab, sp2_tab):
        return pl.pallas_call(
            _tail_body,
            grid=(_HB,),
            in_specs=[pl.BlockSpec((_N, 256), lambda b: (b, 0)),
                      pl.BlockSpec((_N, 256), lambda b: (b, 0)),
                      pl.BlockSpec((_N, 256), lambda b: (b, 0)),
                      pl.BlockSpec((_N, 512), lambda b: (b, 0))] + [_wspec(w) for w in tw],
            out_specs=pl.BlockSpec((1, 1, 1024), lambda b: (b, 0, 0)),
            out_shape=jax.ShapeDtypeStruct((_HB, 1, 1024), jnp.float32),
            compiler_params=pltpu.CompilerParams(dimension_semantics=("arbitrary",)),
        )(gm1, st1_tab, sp1_tab, sp2_tab, *tw)

    _gmax = _make_gather_max3(256, _HR)

    st0_0, sp1_0 = front(0)
    st0_1, sp1_1 = front(1)
    gm0_0 = _gmax(st0_0, *gih[0])
    gm0_1 = _gmax(st0_1, *gih[1])
    st1_0, sp2_0 = mid(gm0_0, st0_0, sp1_0)
    gm1_0 = _gmax(st1_0, *gih[0])
    st1_1, sp2_1 = mid(gm0_1, st0_1, sp1_1)
    gm1_1 = _gmax(st1_1, *gih[1])
    fm0 = tail(gm1_0, st1_0, sp1_0, sp2_0)
    fm1 = tail(gm1_1, st1_1, sp1_1, sp2_1)
    fm = jnp.concatenate([fm0, fm1], axis=0).reshape(_B, 1024)

    hw = [p['l1_W'].T, p['l1_b'][None, :],
          p['l2_W'].T, p['l2_b'][None, :],
          p['l3_W'].T, p['l3_b'][None, :]]
    y, hn = pl.pallas_call(
        _head_body,
        in_specs=[pl.BlockSpec(fm.shape, lambda: (0, 0))] +
                 [pl.BlockSpec(w.shape, lambda: (0, 0)) for w in hw],
        out_specs=[pl.BlockSpec((_B, 40), lambda: (0, 0)),
                   pl.BlockSpec((_B, 256), lambda: (0, 0))],
        out_shape=[jax.ShapeDtypeStruct((_B, 40), jnp.float32),
                   jax.ShapeDtypeStruct((_B, 256), jnp.float32)],
    )(fm, *hw)
    return y, hn


# speedup vs baseline: 52.2985x; 1.0798x over previous
import functools

import jax
import jax.numpy as jnp
import numpy as np
from jax import lax
from jax.experimental import pallas as pl
from jax.experimental.pallas import tpu as pltpu
from jax.experimental.pallas import tpu_sc as plsc

_EPS = 1e-5
_N = 1024
_B = 16
_NW = 32
_INV2SIG2 = 12.5
_KBN = np.float32(1.0 / np.sqrt(1.0 + _EPS))

_HB = _B // 2
_HR = _HB * _N


def _mm(a, b):
    return jax.lax.dot_general(a, b, (((1,), (0,)), ((), ())),
                               preferred_element_type=jnp.float32)


def _mmT(a, w):
    return jax.lax.dot_general(a, w, (((1,), (1,)), ((), ())),
                               preferred_element_type=jnp.float32)


def _relu(v):
    return jnp.maximum(v, 0.0)


def _blk(x, W, b, g, be):
    z = _mmT(x, W[...])
    return _relu((z + b[...]) * (g[...] * _KBN) + be[...])


_CS = 32


def _make_gather_max3(D, R):
    per_w = R // _NW
    n_sub = per_w // _CS
    mesh = plsc.VectorSubcoreMesh(core_axis_name="c", subcore_axis_name="s",
                                  num_cores=2, num_subcores=16)
    row_bufs = [pltpu.VMEM((_CS, D), jnp.float32) for _ in range(6)]

    @functools.partial(
        pl.kernel, mesh=mesh,
        out_type=jax.ShapeDtypeStruct((R, D), jnp.float32),
        scratch_types=[
            pltpu.VMEM((n_sub, _CS), jnp.int32),
            pltpu.VMEM((n_sub, _CS), jnp.int32),
            pltpu.VMEM((n_sub, _CS), jnp.int32),
        ] + row_bufs + [pltpu.SemaphoreType.DMA, pltpu.SemaphoreType.DMA],
    )
    def gmax(tab, i0, i1, i2, out,
             iv0, iv1, iv2, ga0, ga1, ga2, gb0, gb1, gb2, sa, sb):
        wid = lax.axis_index("s") * 2 + lax.axis_index("c")
        pltpu.sync_copy(i0.at[pl.ds(wid * n_sub, n_sub)], iv0)
        pltpu.sync_copy(i1.at[pl.ds(wid * n_sub, n_sub)], iv1)
        pltpu.sync_copy(i2.at[pl.ds(wid * n_sub, n_sub)], iv2)
        stages = [((ga0, ga1, ga2), sa), ((gb0, gb1, gb2), sb)]

        def issue(j, stage):
            bufs, sem = stage
            return [pltpu.async_copy(tab.at[iv.at[j]], buf, sem)
                    for iv, buf in zip((iv0, iv1, iv2), bufs)]

        def crunch(j, stage, cps):
            (b0, b1, b2), _ = stage
            for cp in cps:
                cp.wait()

            def row(r, carry):
                for c in range(D // 16):
                    sl = pl.ds(c * 16, 16)
                    b0[r, sl] = jnp.maximum(jnp.maximum(b0[r, sl], b1[r, sl]),
                                            b2[r, sl])
                return carry

            lax.fori_loop(0, _CS, row, 0)
            pltpu.sync_copy(b0, out.at[pl.ds(wid * per_w + j * _CS, _CS)])

        pend = issue(0, stages[0])
        for j in range(1, n_sub):
            nxt = issue(j, stages[j % 2])
            crunch(j - 1, stages[(j - 1) % 2], pend)
            pend = nxt
        crunch(n_sub - 1, stages[(n_sub - 1) % 2], pend)

    return gmax


def _gather(P, M):
    hi = M.astype(jnp.bfloat16)
    lo = (M - hi.astype(jnp.float32)).astype(jnp.bfloat16)
    return _mm(P, hi) + _mm(P, lo)


def _front_body(face_ref, nbr_ref,
                sp1W, sp1b, sp1g, sp1be, sp2W, sp2b, sp2g, sp2be,
                r1W, r1b, r1g, r1be, r2W, r2b, r2g, r2be,
                f1W, f1b, f1g, f1be, f2W, f2b, f2g, f2be,
                Wm, fkg, fkbe,
                s1Wa, s1Wb, s1Wc, s1b, s1g, s1be,
                s2Wp, s2bp, s2gp, s2bep,
                c1Wa, c1Wb, c1b, c1g, c1be,
                st0_ref, sp1_ref):
    f = face_ref[0]
    centers = f[:, 0:3]
    corners = f[:, 3:12] - jnp.concatenate([centers, centers, centers], axis=1)
    nrm = f[:, 12:15]

    sf0 = _blk(_blk(centers, sp1W, sp1b, sp1g, sp1be), sp2W, sp2b, sp2g, sp2be)

    def rmlp(c):
        return _blk(_blk(c, r1W, r1b, r1g, r1be), r2W, r2b, r2g, r2be)
    fea = (rmlp(corners[:, 0:6]) + rmlp(corners[:, 3:9]) +
           rmlp(jnp.concatenate([corners[:, 6:9], corners[:, 0:3]], axis=1))) * (1.0 / 3.0)
    frc = _blk(_blk(fea, f1W, f1b, f1g, f1be), f2W, f2b, f2g, f2be)

    nbr = nbr_ref[0]
    lane = jax.lax.broadcasted_iota(jnp.int32, (_N, _N), 1)
    Wmv = Wm[...]
    wsq = jnp.sum(Wmv * Wmv, axis=0, keepdims=True)
    acc = jnp.zeros((_N, 256), jnp.float32)
    for F in (nrm,
              _gather((lane == nbr[:, 0:1]).astype(jnp.bfloat16), nrm),
              _gather((lane == nbr[:, 1:2]).astype(jnp.bfloat16), nrm),
              _gather((lane == nbr[:, 2:3]).astype(jnp.bfloat16), nrm)):
        fsq = jnp.sum(F * F, axis=1, keepdims=True)
        d = fsq + wsq - 2.0 * _mm(F, Wmv)
        acc = acc + jnp.exp(-_INV2SIG2 * d)
    fkc = (acc[:, 0:64] + acc[:, 64:128] + acc[:, 128:192] + acc[:, 192:256]) * (1.0 / 16.0)
    fkc = _relu(fkc * (fkg[...] * _KBN) + fkbe[...])

    st = _relu((_mmT(frc, s1Wa[...]) + _mmT(fkc, s1Wb[...]) +
                _mmT(nrm, s1Wc[...]) + s1b[...]) * (s1g[...] * _KBN) + s1be[...])
    st0 = _relu((_mmT(st, s2Wp[...]) + s2bp[...]) * (s2gp[...] * _KBN) + s2bep[...])
    st0_ref[...] = st0
    sp1_ref[...] = _relu((_mmT(sf0, c1Wa[...]) + _mmT(st0, c1Wb[...]) +
                          c1b[...]) * (c1g[...] * _KBN) + c1be[...])


def _mid_body(gm0_ref, st0_ref, sp1_ref,
              a1Wp, a1b, a1g, a1be, c2W, c2b, c2g, c2be,
              st1_ref, sp2_ref):
    st1 = _blk(jnp.maximum(gm0_ref[...], st0_ref[...]), a1Wp, a1b, a1g, a1be)
    st1_ref[...] = st1
    sp2_ref[...] = _blk(jnp.concatenate([sp1_ref[...], st1], axis=1),
                        c2W, c2b, c2g, c2be)


def _tail_body(gm1_ref, st1_ref, sp1_ref, sp2_ref,
               a2W, a2b, a2g, a2be, fuW, fub, fug, fube,
               caW, cab, cag, cabe, out_ref):
    sp2 = sp2_ref[...]
    st2 = _blk(jnp.maximum(gm1_ref[...], st1_ref[...]), a2W, a2b, a2g, a2be)
    sp3 = _blk(jnp.concatenate([sp2, st2], axis=1), fuW, fub, fug, fube)
    fea2 = _blk(jnp.concatenate([sp1_ref[...], sp2, sp3], axis=1),
                caW, cab, cag, cabe)
    out_ref[0] = jnp.max(fea2, axis=0, keepdims=True)


def _head_body(fm_ref, W1, b1, W2, b2, W3, b3, y_ref, hn_ref):
    h = _relu(_mmT(fm_ref[...], W1[...]) + b1[...])
    h = _relu(_mmT(h, W2[...]) + b2[...])
    y_ref[...] = _mmT(h, W3[...]) + b3[...]
    hn_ref[...] = h * jax.lax.rsqrt(jnp.sum(h * h))


def _wspec(w):
    return pl.BlockSpec(w.shape, functools.partial(lambda b, _s: (0,) * len(_s), _s=w.shape))


@jax.jit
def kernel(x, params):
    p = params
    face = x[..., :15]
    nbr = x[..., 15:18].astype(jnp.int32)
    gih = []
    for h in range(2):
        gidx = (nbr[h * _HB:(h + 1) * _HB] +
                (jnp.arange(_HB, dtype=jnp.int32) * _N)[:, None, None]).reshape(_HR, 3)
        gih.append(tuple(gidx[:, k].reshape(_HR // _CS, _CS) for k in range(3)))

    def b3(name):
        return (p[name + '_b'][None, :], p[name + '_g'][None, :],
                p[name + '_be'][None, :])

    alpha, beta = p['fkc_alpha'], p['fkc_beta']
    w3 = jnp.concatenate([jnp.sin(alpha) * jnp.cos(beta),
                          jnp.sin(alpha) * jnp.sin(beta),
                          jnp.cos(alpha)], axis=0)
    Wm = jnp.transpose(w3, (0, 2, 1)).reshape(3, 256)

    s1W = p['str1_W']
    s2Wp = jnp.pad(p['str2_W'], ((0, 125), (0, 0)))
    s2bp = jnp.pad(p['str2_b'], (0, 125))[None, :]
    s2gp = jnp.pad(p['str2_g'], (0, 125))[None, :]
    s2bep = jnp.pad(p['str2_be'], (0, 125))[None, :]
    c1W = p['mc1_comb_W']
    c1Wb = jnp.pad(c1W[:, 64:195], ((0, 0), (0, 125)))
    a1Wp = jnp.pad(p['mc1_agg_W'], ((0, 0), (0, 125)))

    fw = [p['sp1_W'], *b3('sp1'), p['sp2_W'], *b3('sp2'),
          p['rot1_W'], *b3('rot1'), p['rot2_W'], *b3('rot2'),
          p['fus1_W'], *b3('fus1'), p['fus2_W'], *b3('fus2'),
          Wm, p['fkc_g'][None, :], p['fkc_be'][None, :],
          s1W[:, 0:64], s1W[:, 64:128], s1W[:, 128:131], *b3('str1'),
          s2Wp, s2bp, s2gp, s2bep,
          c1W[:, 0:64], c1Wb, *b3('mc1_comb')]
    mw = [a1Wp, *b3('mc1_agg'), p['mc2_comb_W'], *b3('mc2_comb')]
    tw = [p['mc2_agg_W'], *b3('mc2_agg'), p['fusion_W'], *b3('fusion'),
          p['cat_W'], *b3('cat')]

    def front(h):
        return pl.pallas_call(
            _front_body,
            grid=(_HB,),
            in_specs=[pl.BlockSpec((1, _N, 15), lambda b: (b, 0, 0)),
                      pl.BlockSpec((1, _N, 3), lambda b: (b, 0, 0))] + [_wspec(w) for w in fw],
            out_specs=[pl.BlockSpec((_N, 256), lambda b: (b, 0)),
                       pl.BlockSpec((_N, 256), lambda b: (b, 0))],
            out_shape=[jax.ShapeDtypeStruct((_HR, 256), jnp.float32),
                       jax.ShapeDtypeStruct((_HR, 256), jnp.float32)],
            compiler_params=pltpu.CompilerParams(dimension_semantics=("arbitrary",)),
        )(face[h * _HB:(h + 1) * _HB], nbr[h * _HB:(h + 1) * _HB], *fw)

    def mid(gm0, st0_tab, sp1_tab):
        return pl.pallas_call(
            _mid_body,
            grid=(_HB,),
            in_specs=[pl.BlockSpec((_N, 256), lambda b: (b, 0)),
                      pl.BlockSpec((_N, 256), lambda b: (b, 0)),
                      pl.BlockSpec((_N, 256), lambda b: (b, 0))] + [_wspec(w) for w in mw],
            out_specs=[pl.BlockSpec((_N, 256), lambda b: (b, 0)),
                       pl.BlockSpec((_N, 512), lambda b: (b, 0))],
            out_shape=[jax.ShapeDtypeStruct((_HR, 256), jnp.float32),
                       jax.ShapeDtypeStruct((_HR, 512), jnp.float32)],
            compiler_params=pltpu.CompilerParams(dimension_semantics=("arbitrary",)),
        )(gm0, st0_tab, sp1_tab, *mw)

    def tail(gm1, st1_tab, sp1_tab, sp2_tab):
        return pl.pallas_call(
            _tail_body,
            grid=(_HB,),
            in_specs=[pl.BlockSpec((_N, 256), lambda b: (b, 0)),
                      pl.BlockSpec((_N, 256), lambda b: (b, 0)),
                      pl.BlockSpec((_N, 256), lambda b: (b, 0)),
                      pl.BlockSpec((_N, 512), lambda b: (b, 0))] + [_wspec(w) for w in tw],
            out_specs=pl.BlockSpec((1, 1, 1024), lambda b: (b, 0, 0)),
            out_shape=jax.ShapeDtypeStruct((_HB, 1, 1024), jnp.float32),
            compiler_params=pltpu.CompilerParams(dimension_semantics=("arbitrary",)),
        )(gm1, st1_tab, sp1_tab, sp2_tab, *tw)

    _gmax = _make_gather_max3(256, _HR)

    st0_0, sp1_0 = front(0)
    st0_1, sp1_1 = front(1)
    gm0_0 = _gmax(st0_0, *gih[0])
    gm0_1 = _gmax(st0_1, *gih[1])
    st1_0, sp2_0 = mid(gm0_0, st0_0, sp1_0)
    gm1_0 = _gmax(st1_0, *gih[0])
    st1_1, sp2_1 = mid(gm0_1, st0_1, sp1_1)
    gm1_1 = _gmax(st1_1, *gih[1])
    fm0 = tail(gm1_0, st1_0, sp1_0, sp2_0)
    fm1 = tail(gm1_1, st1_1, sp1_1, sp2_1)
    fm = jnp.concatenate([fm0, fm1], axis=0).reshape(_B, 1024)

    hw = [p['l1_W'], p['l1_b'][None, :],
          p['l2_W'], p['l2_b'][None, :],
          p['l3_W'], p['l3_b'][None, :]]
    y, hn = pl.pallas_call(
        _head_body,
        in_specs=[pl.BlockSpec(fm.shape, lambda: (0, 0))] +
                 [pl.BlockSpec(w.shape, lambda: (0, 0)) for w in hw],
        out_specs=[pl.BlockSpec((_B, 40), lambda: (0, 0)),
                   pl.BlockSpec((_B, 256), lambda: (0, 0))],
        out_shape=[jax.ShapeDtypeStruct((_B, 40), jnp.float32),
                   jax.ShapeDtypeStruct((_B, 256), jnp.float32)],
    )(fm, *hw)
    return y, hn


# speedup vs baseline: 58.1510x; 1.1119x over previous
import functools

import jax
import jax.numpy as jnp
import numpy as np
from jax import lax
from jax.experimental import pallas as pl
from jax.experimental.pallas import tpu as pltpu
from jax.experimental.pallas import tpu_sc as plsc

_EPS = 1e-5
_N = 1024
_B = 16
_NW = 32
_INV2SIG2 = 12.5
_KBN = np.float32(1.0 / np.sqrt(1.0 + _EPS))

_HB = _B // 2
_HR = _HB * _N


def _mm(a, b):
    return jax.lax.dot_general(a, b, (((1,), (0,)), ((), ())),
                               preferred_element_type=jnp.float32)


def _mmT(a, w):
    return jax.lax.dot_general(a, w, (((1,), (1,)), ((), ())),
                               preferred_element_type=jnp.float32)


def _relu(v):
    return jnp.maximum(v, 0.0)


def _blk(x, W, b, g, be):
    z = _mmT(x, W[...])
    return _relu((z + b[...]) * (g[...] * _KBN) + be[...])


_CS = 32


def _make_gather_max3(D, R):
    per_w = R // _NW
    n_sub = per_w // _CS
    mesh = plsc.VectorSubcoreMesh(core_axis_name="c", subcore_axis_name="s",
                                  num_cores=2, num_subcores=16)
    row_bufs = [pltpu.VMEM((_CS, D), jnp.float32) for _ in range(6)]

    @functools.partial(
        pl.kernel, mesh=mesh,
        out_type=jax.ShapeDtypeStruct((R, D), jnp.float32),
        scratch_types=[
            pltpu.VMEM((n_sub, _CS), jnp.int32),
            pltpu.VMEM((n_sub, _CS), jnp.int32),
            pltpu.VMEM((n_sub, _CS), jnp.int32),
        ] + row_bufs + [pltpu.SemaphoreType.DMA, pltpu.SemaphoreType.DMA],
    )
    def gmax(tab, i0, i1, i2, out,
             iv0, iv1, iv2, ga0, ga1, ga2, gb0, gb1, gb2, sa, sb):
        wid = lax.axis_index("s") * 2 + lax.axis_index("c")
        pltpu.sync_copy(i0.at[pl.ds(wid * n_sub, n_sub)], iv0)
        pltpu.sync_copy(i1.at[pl.ds(wid * n_sub, n_sub)], iv1)
        pltpu.sync_copy(i2.at[pl.ds(wid * n_sub, n_sub)], iv2)
        stages = [((ga0, ga1, ga2), sa), ((gb0, gb1, gb2), sb)]

        def issue(j, stage):
            bufs, sem = stage
            return [pltpu.async_copy(tab.at[iv.at[j]], buf, sem)
                    for iv, buf in zip((iv0, iv1, iv2), bufs)]

        def crunch(j, stage, cps):
            (b0, b1, b2), _ = stage
            for cp in cps:
                cp.wait()

            def row(r, carry):
                for c in range(D // 16):
                    sl = pl.ds(c * 16, 16)
                    b0[r, sl] = jnp.maximum(jnp.maximum(b0[r, sl], b1[r, sl]),
                                            b2[r, sl])
                return carry

            lax.fori_loop(0, _CS, row, 0)
            pltpu.sync_copy(b0, out.at[pl.ds(wid * per_w + j * _CS, _CS)])

        pend = issue(0, stages[0])
        for j in range(1, n_sub):
            nxt = issue(j, stages[j % 2])
            crunch(j - 1, stages[(j - 1) % 2], pend)
            pend = nxt
        crunch(n_sub - 1, stages[(n_sub - 1) % 2], pend)

    return gmax


def _gather(P, M):
    hi = M.astype(jnp.bfloat16)
    lo = (M - hi.astype(jnp.float32)).astype(jnp.bfloat16)
    return _mm(P, hi) + _mm(P, lo)


def _front_body(x_ref,
                sp1W, sp1b, sp1g, sp1be, sp2W, sp2b, sp2g, sp2be,
                r1W, r1b, r1g, r1be, r2W, r2b, r2g, r2be,
                f1W, f1b, f1g, f1be, f2W, f2b, f2g, f2be,
                Wm, fkg, fkbe,
                s1Wa, s1Wb, s1Wc, s1b, s1g, s1be,
                s2Wp, s2bp, s2gp, s2bep,
                c1Wa, c1Wb, c1b, c1g, c1be,
                st0_ref, sp1_ref):
    f = x_ref[0]
    centers = f[:, 0:3]
    corners = f[:, 3:12] - jnp.concatenate([centers, centers, centers], axis=1)
    nrm = f[:, 12:15]

    sf0 = _blk(_blk(centers, sp1W, sp1b, sp1g, sp1be), sp2W, sp2b, sp2g, sp2be)

    def rmlp(c):
        return _blk(_blk(c, r1W, r1b, r1g, r1be), r2W, r2b, r2g, r2be)
    fea = (rmlp(corners[:, 0:6]) + rmlp(corners[:, 3:9]) +
           rmlp(jnp.concatenate([corners[:, 6:9], corners[:, 0:3]], axis=1))) * (1.0 / 3.0)
    frc = _blk(_blk(fea, f1W, f1b, f1g, f1be), f2W, f2b, f2g, f2be)

    nbr = f[:, 15:18].astype(jnp.int32)
    lane = jax.lax.broadcasted_iota(jnp.int32, (_N, _N), 1)
    Wmv = Wm[...]
    wsq = jnp.sum(Wmv * Wmv, axis=0, keepdims=True)
    acc = jnp.zeros((_N, 256), jnp.float32)
    for F in (nrm,
              _gather((lane == nbr[:, 0:1]).astype(jnp.bfloat16), nrm),
              _gather((lane == nbr[:, 1:2]).astype(jnp.bfloat16), nrm),
              _gather((lane == nbr[:, 2:3]).astype(jnp.bfloat16), nrm)):
        fsq = jnp.sum(F * F, axis=1, keepdims=True)
        d = fsq + wsq - 2.0 * _mm(F, Wmv)
        acc = acc + jnp.exp(-_INV2SIG2 * d)
    ridx = jax.lax.broadcasted_iota(jnp.int32, (256, 64), 0)
    cidx = jax.lax.broadcasted_iota(jnp.int32, (256, 64), 1)
    S = ((ridx // 4) == cidx).astype(jnp.bfloat16)
    hi = acc.astype(jnp.bfloat16)
    lo = (acc - hi.astype(jnp.float32)).astype(jnp.bfloat16)
    fkc = (_mm(hi, S) + _mm(lo, S)) * (1.0 / 16.0)
    fkc = _relu(fkc * (fkg[...] * _KBN) + fkbe[...])

    st = _relu((_mmT(frc, s1Wa[...]) + _mmT(fkc, s1Wb[...]) +
                _mmT(nrm, s1Wc[...]) + s1b[...]) * (s1g[...] * _KBN) + s1be[...])
    st0 = _relu((_mmT(st, s2Wp[...]) + s2bp[...]) * (s2gp[...] * _KBN) + s2bep[...])
    st0_ref[...] = st0
    sp1_ref[...] = _relu((_mmT(sf0, c1Wa[...]) + _mmT(st0, c1Wb[...]) +
                          c1b[...]) * (c1g[...] * _KBN) + c1be[...])


def _mid_body(gm0_ref, st0_ref, sp1_ref,
              a1Wp, a1b, a1g, a1be, c2W, c2b, c2g, c2be,
              st1_ref, sp2_ref):
    st1 = _blk(jnp.maximum(gm0_ref[...], st0_ref[...]), a1Wp, a1b, a1g, a1be)
    st1_ref[...] = st1
    sp2_ref[...] = _blk(jnp.concatenate([sp1_ref[...], st1], axis=1),
                        c2W, c2b, c2g, c2be)


def _tail_body(gm1_ref, st1_ref, sp1_ref, sp2_ref,
               a2W, a2b, a2g, a2be, fuW, fub, fug, fube,
               caW, cab, cag, cabe, out_ref):
    sp2 = sp2_ref[...]
    st2 = _blk(jnp.maximum(gm1_ref[...], st1_ref[...]), a2W, a2b, a2g, a2be)
    sp3 = _blk(jnp.concatenate([sp2, st2], axis=1), fuW, fub, fug, fube)
    fea2 = _blk(jnp.concatenate([sp1_ref[...], sp2, sp3], axis=1),
                caW, cab, cag, cabe)
    out_ref[0] = jnp.max(fea2, axis=0, keepdims=True)


def _head_body(fm_ref, W1, b1, W2, b2, W3, b3, y_ref, hn_ref):
    h = _relu(_mmT(fm_ref[...], W1[...]) + b1[...])
    h = _relu(_mmT(h, W2[...]) + b2[...])
    y_ref[...] = _mmT(h, W3[...]) + b3[...]
    hn_ref[...] = h * jax.lax.rsqrt(jnp.sum(h * h))


def _wspec(w):
    return pl.BlockSpec(w.shape, functools.partial(lambda b, _s: (0,) * len(_s), _s=w.shape))


@jax.jit
def kernel(x, params):
    p = params
    offs = (jnp.arange(_HB, dtype=jnp.float32) * _N)[:, None, None]
    gih = []
    for h in range(2):
        gidx = (x[h * _HB:(h + 1) * _HB, :, 15:18] + offs).astype(jnp.int32)
        gidx = gidx.reshape(_HR, 3)
        gih.append(tuple(gidx[:, k].reshape(_HR // _CS, _CS) for k in range(3)))

    def b3(name):
        return (p[name + '_b'][None, :], p[name + '_g'][None, :],
                p[name + '_be'][None, :])

    alpha, beta = p['fkc_alpha'], p['fkc_beta']
    w3 = jnp.concatenate([jnp.sin(alpha) * jnp.cos(beta),
                          jnp.sin(alpha) * jnp.sin(beta),
                          jnp.cos(alpha)], axis=0)
    Wm = w3.reshape(3, 256)

    s1W = p['str1_W']
    s2Wp = jnp.pad(p['str2_W'], ((0, 125), (0, 0)))
    s2bp = jnp.pad(p['str2_b'], (0, 125))[None, :]
    s2gp = jnp.pad(p['str2_g'], (0, 125))[None, :]
    s2bep = jnp.pad(p['str2_be'], (0, 125))[None, :]
    c1W = p['mc1_comb_W']
    c1Wb = jnp.pad(c1W[:, 64:195], ((0, 0), (0, 125)))
    a1Wp = jnp.pad(p['mc1_agg_W'], ((0, 0), (0, 125)))

    fw = [p['sp1_W'], *b3('sp1'), p['sp2_W'], *b3('sp2'),
          p['rot1_W'], *b3('rot1'), p['rot2_W'], *b3('rot2'),
          p['fus1_W'], *b3('fus1'), p['fus2_W'], *b3('fus2'),
          Wm, p['fkc_g'][None, :], p['fkc_be'][None, :],
          s1W[:, 0:64], s1W[:, 64:128], s1W[:, 128:131], *b3('str1'),
          s2Wp, s2bp, s2gp, s2bep,
          c1W[:, 0:64], c1Wb, *b3('mc1_comb')]
    mw = [a1Wp, *b3('mc1_agg'), p['mc2_comb_W'], *b3('mc2_comb')]
    tw = [p['mc2_agg_W'], *b3('mc2_agg'), p['fusion_W'], *b3('fusion'),
          p['cat_W'], *b3('cat')]

    def front(h):
        return pl.pallas_call(
            _front_body,
            grid=(_HB,),
            in_specs=[pl.BlockSpec((1, _N, 18),
                                   functools.partial(lambda b, _h: (b + _h, 0, 0),
                                                     _h=h * _HB))] + [_wspec(w) for w in fw],
            out_specs=[pl.BlockSpec((_N, 256), lambda b: (b, 0)),
                       pl.BlockSpec((_N, 256), lambda b: (b, 0))],
            out_shape=[jax.ShapeDtypeStruct((_HR, 256), jnp.float32),
                       jax.ShapeDtypeStruct((_HR, 256), jnp.float32)],
            compiler_params=pltpu.CompilerParams(dimension_semantics=("arbitrary",)),
        )(x, *fw)

    def mid(gm0, st0_tab, sp1_tab):
        return pl.pallas_call(
            _mid_body,
            grid=(_HB,),
            in_specs=[pl.BlockSpec((_N, 256), lambda b: (b, 0)),
                      pl.BlockSpec((_N, 256), lambda b: (b, 0)),
                      pl.BlockSpec((_N, 256), lambda b: (b, 0))] + [_wspec(w) for w in mw],
            out_specs=[pl.BlockSpec((_N, 256), lambda b: (b, 0)),
                       pl.BlockSpec((_N, 512), lambda b: (b, 0))],
            out_shape=[jax.ShapeDtypeStruct((_HR, 256), jnp.float32),
                       jax.ShapeDtypeStruct((_HR, 512), jnp.float32)],
            compiler_params=pltpu.CompilerParams(dimension_semantics=("arbitrary",)),
        )(gm0, st0_tab, sp1_tab, *mw)

    def tail(gm1, st1_tab, sp1_tab, sp2_tab):
        return pl.pallas_call(
            _tail_body,
            grid=(_HB,),
            in_specs=[pl.BlockSpec((_N, 256), lambda b: (b, 0)),
                      pl.BlockSpec((_N, 256), lambda b: (b, 0)),
                      pl.BlockSpec((_N, 256), lambda b: (b, 0)),
                      pl.BlockSpec((_N, 512), lambda b: (b, 0))] + [_wspec(w) for w in tw],
            out_specs=pl.BlockSpec((1, 1, 1024), lambda b: (b, 0, 0)),
            out_shape=jax.ShapeDtypeStruct((_HB, 1, 1024), jnp.float32),
            compiler_params=pltpu.CompilerParams(dimension_semantics=("arbitrary",)),
        )(gm1, st1_tab, sp1_tab, sp2_tab, *tw)

    _gmax = _make_gather_max3(256, _HR)

    st0_0, sp1_0 = front(0)
    st0_1, sp1_1 = front(1)
    gm0_0 = _gmax(st0_0, *gih[0])
    gm0_1 = _gmax(st0_1, *gih[1])
    st1_0, sp2_0 = mid(gm0_0, st0_0, sp1_0)
    gm1_0 = _gmax(st1_0, *gih[0])
    st1_1, sp2_1 = mid(gm0_1, st0_1, sp1_1)
    gm1_1 = _gmax(st1_1, *gih[1])
    fm0 = tail(gm1_0, st1_0, sp1_0, sp2_0)
    fm1 = tail(gm1_1, st1_1, sp1_1, sp2_1)
    fm = jnp.concatenate([fm0, fm1], axis=0).reshape(_B, 1024)

    hw = [p['l1_W'], p['l1_b'][None, :],
          p['l2_W'], p['l2_b'][None, :],
          p['l3_W'], p['l3_b'][None, :]]
    y, hn = pl.pallas_call(
        _head_body,
        in_specs=[pl.BlockSpec(fm.shape, lambda: (0, 0))] +
                 [pl.BlockSpec(w.shape, lambda: (0, 0)) for w in hw],
        out_specs=[pl.BlockSpec((_B, 40), lambda: (0, 0)),
                   pl.BlockSpec((_B, 256), lambda: (0, 0))],
        out_shape=[jax.ShapeDtypeStruct((_B, 40), jnp.float32),
                   jax.ShapeDtypeStruct((_B, 256), jnp.float32)],
    )(fm, *hw)
    return y, hn
